# Initial kernel scaffold; baseline (speedup 1.0000x reference)
import jax
import jax.numpy as jnp
from jax.experimental import pallas as pl


def kernel(inputs, targets):
    raise NotImplementedError("write your pallas kernel here")


# speedup vs baseline: 2.2094x; 2.2094x over previous
import jax
import jax.numpy as jnp
from jax.experimental import pallas as pl
from jax.experimental.pallas import tpu as pltpu


def _body(x_ref, t_ref, out_ref):
    pid = pl.program_id(0)
    x = x_ref[...]
    t = t_ref[0, 0, :]
    blk, c = x.shape

    e = jnp.exp(x)
    s = jnp.sum(e, axis=1)

    col = jax.lax.broadcasted_iota(jnp.int32, (blk, c), 1)
    xt = jnp.sum(jnp.where(col == t[:, None], x, 0.0), axis=1)
    x0 = x[:, 0]

    d = x0 - xt
    sp = jnp.maximum(d, 0.0) + jnp.log(1.0 + jnp.exp(-jnp.abs(d)))
    loss_rows = jnp.where(t == 0, jnp.log(s) - x0, sp)
    partial = jnp.sum(loss_rows)

    @pl.when(pid == 0)
    def _():
        out_ref[0, 0] = 0.0

    out_ref[0, 0] += partial


def kernel(inputs, targets):
    B, C = inputs.shape
    BLK = 256
    grid = B // BLK
    t3 = targets.astype(jnp.int32).reshape(grid, 1, BLK)

    out = pl.pallas_call(
        _body,
        grid=(grid,),
        in_specs=[
            pl.BlockSpec((BLK, C), lambda i: (i, 0)),
            pl.BlockSpec((1, 1, BLK), lambda i: (i, 0, 0)),
        ],
        out_specs=pl.BlockSpec(memory_space=pltpu.SMEM),
        out_shape=jax.ShapeDtypeStruct((1, 1), jnp.float32),
    )(inputs, t3)
    return out[0, 0] / B
